# Initial kernel scaffold; baseline (speedup 1.0000x reference)
import jax
import jax.numpy as jnp
from jax.experimental import pallas as pl


def kernel(queries, memory_bank):
    raise NotImplementedError("write your pallas kernel here")


# speedup vs baseline: 4.4842x; 4.4842x over previous
import jax
import jax.numpy as jnp
from jax.experimental import pallas as pl
from jax.experimental.pallas import tpu as pltpu

Q = 1024
D = 64
B = 16
PPI = 64
KBLK = 2000


def _knn_kernel(q_ref, m_ref, patch_ref, img_ref, acc_ref):
    k = pl.program_id(0)
    nk = pl.num_programs(0)

    q = q_ref[...]
    m = m_ref[...]

    qm = jax.lax.dot_general(q, m, (((1,), (1,)), ((), ())),
                             preferred_element_type=jnp.float32)
    msq = jax.lax.dot_general(jnp.ones((1, D), jnp.float32), m * m,
                              (((1,), (1,)), ((), ())),
                              preferred_element_type=jnp.float32)
    s = msq - 2.0 * qm
    blk_min = jnp.min(s, axis=1, keepdims=True)

    @pl.when(k == 0)
    def _():
        acc_ref[...] = blk_min

    @pl.when(k > 0)
    def _():
        acc_ref[...] = jnp.minimum(acc_ref[...], blk_min)

    @pl.when(k == nk - 1)
    def _():
        qsq = jnp.sum(q * q, axis=1, keepdims=True)
        dmin = jnp.maximum(acc_ref[...] + qsq, 0.0)
        ps = jnp.sqrt(jnp.maximum(dmin, 1e-12))
        patch_ref[...] = ps
        qimg = jax.lax.broadcasted_iota(jnp.int32, (Q, B), 0) // PPI
        img = jax.lax.broadcasted_iota(jnp.int32, (Q, B), 1)
        masked = jnp.where(qimg == img, ps, -jnp.inf)
        img_ref[...] = jnp.max(masked, axis=0, keepdims=True)


def kernel(queries, memory_bank):
    K = memory_bank.shape[0]
    nblk = K // KBLK
    patch, img = pl.pallas_call(
        _knn_kernel,
        grid=(nblk,),
        in_specs=[
            pl.BlockSpec((Q, D), lambda k: (0, 0)),
            pl.BlockSpec((KBLK, D), lambda k: (k, 0)),
        ],
        out_specs=[
            pl.BlockSpec((Q, 1), lambda k: (0, 0)),
            pl.BlockSpec((1, B), lambda k: (0, 0)),
        ],
        out_shape=[
            jax.ShapeDtypeStruct((Q, 1), jnp.float32),
            jax.ShapeDtypeStruct((1, B), jnp.float32),
        ],
        scratch_shapes=[pltpu.VMEM((Q, 1), jnp.float32)],
    )(queries, memory_bank)
    return patch.reshape(Q), img.reshape(B)


# speedup vs baseline: 7.0822x; 1.5794x over previous
import jax
import jax.numpy as jnp
from jax.experimental import pallas as pl
from jax.experimental.pallas import tpu as pltpu

Q = 1024
D = 64
B = 16
PPI = 64
KBLK = 2000


def _knn_kernel(q_ref, m_ref, patch_ref, img_ref, acc_ref):
    k = pl.program_id(0)
    nk = pl.num_programs(0)

    q = q_ref[...]
    m = m_ref[...]

    msq = jnp.sum(m * m, axis=1, keepdims=True)
    ma = jnp.concatenate([m, msq], axis=1)
    qa = jnp.concatenate([q * -2.0, jnp.ones((Q, 1), jnp.float32)],
                         axis=1)
    s = jax.lax.dot_general(qa, ma, (((1,), (1,)), ((), ())),
                            preferred_element_type=jnp.float32)
    blk_min = jnp.min(s, axis=1, keepdims=True)

    @pl.when(k == 0)
    def _():
        acc_ref[...] = blk_min

    @pl.when(k > 0)
    def _():
        acc_ref[...] = jnp.minimum(acc_ref[...], blk_min)

    @pl.when(k == nk - 1)
    def _():
        qsq = jnp.sum(q * q, axis=1, keepdims=True)
        dmin = jnp.maximum(acc_ref[...] + qsq, 0.0)
        ps = jnp.sqrt(jnp.maximum(dmin, 1e-12))
        patch_ref[...] = ps
        qimg = jax.lax.broadcasted_iota(jnp.int32, (Q, B), 0) // PPI
        img = jax.lax.broadcasted_iota(jnp.int32, (Q, B), 1)
        masked = jnp.where(qimg == img, ps, -jnp.inf)
        img_ref[...] = jnp.max(masked, axis=0, keepdims=True)


def kernel(queries, memory_bank):
    K = memory_bank.shape[0]
    nblk = K // KBLK
    patch, img = pl.pallas_call(
        _knn_kernel,
        grid=(nblk,),
        in_specs=[
            pl.BlockSpec((Q, D), lambda k: (0, 0)),
            pl.BlockSpec((KBLK, D), lambda k: (k, 0)),
        ],
        out_specs=[
            pl.BlockSpec((Q, 1), lambda k: (0, 0)),
            pl.BlockSpec((1, B), lambda k: (0, 0)),
        ],
        out_shape=[
            jax.ShapeDtypeStruct((Q, 1), jnp.float32),
            jax.ShapeDtypeStruct((1, B), jnp.float32),
        ],
        scratch_shapes=[pltpu.VMEM((Q, 1), jnp.float32)],
    )(queries, memory_bank)
    return patch.reshape(Q), img.reshape(B)


# speedup vs baseline: 7.4622x; 1.0537x over previous
import jax
import jax.numpy as jnp
from jax.experimental import pallas as pl
from jax.experimental.pallas import tpu as pltpu

Q = 1024
D = 64
B = 16
PPI = 64
KBLK = 2000


def _knn_kernel(qt_ref, m_ref, patch_ref, img_ref, acc_ref):
    k = pl.program_id(0)
    nk = pl.num_programs(0)

    qt = qt_ref[...]
    m = m_ref[...]

    msq = jax.lax.dot_general(m * m, jnp.ones((D, 1), jnp.float32),
                              (((1,), (0,)), ((), ())),
                              preferred_element_type=jnp.float32)
    ma = jnp.concatenate([m, msq], axis=1)
    qa = jnp.concatenate([qt * -2.0, jnp.ones((1, Q), jnp.float32)],
                         axis=0)
    s = jax.lax.dot_general(ma, qa, (((1,), (0,)), ((), ())),
                            preferred_element_type=jnp.float32)
    blk_min = jnp.min(s, axis=0, keepdims=True)

    @pl.when(k == 0)
    def _():
        acc_ref[...] = blk_min

    @pl.when(k > 0)
    def _():
        acc_ref[...] = jnp.minimum(acc_ref[...], blk_min)

    @pl.when(k == nk - 1)
    def _():
        qsq = jnp.sum(qt * qt, axis=0, keepdims=True)
        dmin = jnp.maximum(acc_ref[...] + qsq, 0.0)
        ps = jnp.sqrt(jnp.maximum(dmin, 1e-12))
        patch_ref[...] = ps
        qimg = jax.lax.broadcasted_iota(jnp.int32, (B, Q), 1) // PPI
        img = jax.lax.broadcasted_iota(jnp.int32, (B, Q), 0)
        masked = jnp.where(qimg == img, ps, -jnp.inf)
        img_ref[...] = jnp.max(masked, axis=1, keepdims=True)


def kernel(queries, memory_bank):
    K = memory_bank.shape[0]
    nblk = K // KBLK
    qt = queries.T
    patch, img = pl.pallas_call(
        _knn_kernel,
        grid=(nblk,),
        in_specs=[
            pl.BlockSpec((D, Q), lambda k: (0, 0)),
            pl.BlockSpec((KBLK, D), lambda k: (k, 0)),
        ],
        out_specs=[
            pl.BlockSpec((1, Q), lambda k: (0, 0)),
            pl.BlockSpec((B, 1), lambda k: (0, 0)),
        ],
        out_shape=[
            jax.ShapeDtypeStruct((1, Q), jnp.float32),
            jax.ShapeDtypeStruct((B, 1), jnp.float32),
        ],
        scratch_shapes=[pltpu.VMEM((1, Q), jnp.float32)],
    )(qt, memory_bank)
    return patch.reshape(Q), img.reshape(B)
